# baseline (speedup 1.0000x reference)
import functools

import jax
import jax.numpy as jnp
from jax import lax
from jax.experimental import pallas as pl
from jax.experimental.pallas import tpu as pltpu
from jax.experimental.pallas import tpu_sc as plsc

VOCAB = 1000000
EMB_DIM = 64
NUM_Y = 2
BATCH = 4096
HIST = 200

NC = 2
NS = 16
NW = NC * NS

TOTAL_IDX = BATCH * HIST
IDX_PER_W = TOTAL_IDX // NW
CHUNK = 128
CHUNKS_PER_W = IDX_PER_W // CHUNK
BAGS_PER_W = BATCH // NW

PROJ_BLK = 32768
BLOCKS_A = 16
HALF = BLOCKS_A * PROJ_BLK
BLOCKS_B = 15
LEN_A = BLOCKS_A * PROJ_BLK + 8
LEN_B = BLOCKS_B * PROJ_BLK + 8
DUMMY_B = BLOCKS_B * PROJ_BLK


def _make_proj(block_off, nblocks, out_len):
    def body(embt_ref, w_ref, out_hbm, pk_v, z_v, sems):
        i = pl.program_id(0)
        slot = lax.rem(i, 2)
        s = lax.dot_general(
            w_ref[...], embt_ref[...],
            (((1,), (0,)), ((), ())),
            preferred_element_type=jnp.float32,
        )
        s0 = s[0:1, :].astype(jnp.bfloat16)
        s1 = s[1:2, :].astype(jnp.bfloat16)
        u0 = lax.convert_element_type(
            lax.bitcast_convert_type(s0, jnp.uint16), jnp.uint32)
        u1 = lax.convert_element_type(
            lax.bitcast_convert_type(s1, jnp.uint16), jnp.uint32)
        packed = u0 | (u1 << 16)
        pk_v[pl.ds(slot, 1), :] = lax.bitcast_convert_type(packed, jnp.int32)

        def cp(j, sl):
            return pltpu.make_async_copy(
                pk_v.at[sl], out_hbm.at[pl.ds(j * PROJ_BLK, PROJ_BLK)],
                sems.at[sl])

        cp(i, slot).start()

        @pl.when(i > 0)
        def _():
            cp(i - 1, 1 - slot).wait()

        @pl.when(i == nblocks - 1)
        def _():
            z_v[...] = jnp.zeros((1, 8), jnp.int32)
            cz = pltpu.make_async_copy(
                z_v.at[0], out_hbm.at[pl.ds(nblocks * PROJ_BLK, 8)],
                sems.at[2])
            cz.start()
            cz.wait()
            cp(i, slot).wait()

    def call(embT, W):
        return pl.pallas_call(
            body,
            grid=(nblocks,),
            in_specs=[
                pl.BlockSpec((EMB_DIM, PROJ_BLK),
                             lambda i: (0, i + block_off)),
                pl.BlockSpec((NUM_Y, EMB_DIM), lambda i: (0, 0)),
            ],
            out_specs=pl.BlockSpec(memory_space=pl.ANY),
            out_shape=jax.ShapeDtypeStruct((out_len,), jnp.int32),
            scratch_shapes=[pltpu.VMEM((2, PROJ_BLK), jnp.int32),
                            pltpu.VMEM((1, 8), jnp.int32),
                            pltpu.SemaphoreType.DMA((3,))],
        )(embT, W)

    return call


_proj_a = _make_proj(0, BLOCKS_A, LEN_A)
_proj_b = _make_proj(BLOCKS_A, BLOCKS_B, LEN_B)


def _sc_body(final, p_hbm, idx_hbm, *rest):
    if final:
        part_hbm, bias_hbm, out_hbm, idx_v, g_v, o_v, pa_v, b_v, sems = rest
    else:
        out_hbm, idx_v, g_v, o_v, b_v, sems = rest
        part_hbm = pa_v = bias_hbm = None
    wid = lax.axis_index("s") * NC + lax.axis_index("c")
    pltpu.sync_copy(idx_hbm.at[pl.ds(wid * CHUNKS_PER_W, CHUNKS_PER_W)], idx_v)
    if final:
        pltpu.sync_copy(bias_hbm, b_v)
        pltpu.sync_copy(
            part_hbm.at[pl.ds(wid * (BAGS_PER_W * NUM_Y),
                              BAGS_PER_W * NUM_Y)], pa_v)

    def chunk_copy(row, sl):
        return pltpu.make_async_copy(
            p_hbm.at[idx_v.at[row]],
            g_v.at[pl.ds(row * CHUNK, CHUNK)],
            sems.at[sl],
        )

    lanes = lax.iota(jnp.int32, 16)
    zero = jnp.zeros((16,), jnp.float32)
    himask = jnp.int32(-65536)
    dnums = lax.GatherDimensionNumbers(
        offset_dims=(), collapsed_slice_dims=(0,), start_index_map=(0,))
    perms = [(lanes + step) % 16 for step in (8, 4, 2, 1)]
    bias = b_v[...] if final else None

    def lane_perm(x, perm):
        return lax.gather(x, perm[:, None], dnums, (1,),
                          mode=lax.GatherScatterMode.PROMISE_IN_BOUNDS)

    def fold(x):
        for p in perms:
            x = x + lane_perm(x, p)
        return x

    def unpack(w):
        lo = plsc.bitcast(w << 16, jnp.float32)
        hi = plsc.bitcast(w & himask, jnp.float32)
        return lo, hi

    def per_quad(q):
        out_acc = zero
        for t4 in range(4):
            t = q * 4 + t4
            base = t * (2 * HIST)

            def red(k, ab):
                a0, a1 = ab
                lo, hi = unpack(g_v[pl.ds(base + k * 16, 16)])
                return a0 + lo, a1 + hi

            accA0, accA1 = lax.fori_loop(0, 12, red, (zero, zero))
            lo, hi = unpack(g_v[pl.ds(base + 192, 16)])
            mA = lanes < 8
            accA0 = accA0 + jnp.where(mA, lo, zero)
            accA1 = accA1 + jnp.where(mA, hi, zero)
            accB0 = jnp.where(mA, zero, lo)
            accB1 = jnp.where(mA, zero, hi)
            accB0, accB1 = lax.fori_loop(13, 25, red, (accB0, accB1))

            sA0, sA1 = fold(accA0), fold(accA1)
            sB0, sB1 = fold(accB0), fold(accB1)
            qA = jnp.where((lanes & 1) == 0, sA0, sA1)
            qB = jnp.where((lanes & 1) == 0, sB0, sB1)
            quadv = jnp.where((lanes & 2) == 0, qA, qB)
            out_acc = jnp.where((lanes >> 2) == t4, quadv, out_acc)
        if final:
            z = (out_acc + pa_v[pl.ds(q * 16, 16)]) * (1.0 / HIST) + bias
            o_v[pl.ds(q * 16, 16)] = 1.0 / (1.0 + jnp.exp(-z))
        else:
            o_v[pl.ds(q * 16, 16)] = out_acc

    BLK = 25
    NBLK = CHUNKS_PER_W // BLK

    def start_blk(b):
        def go(j, c):
            chunk_copy(b * BLK + j, b & 1).start()
            return c
        lax.fori_loop(0, BLK, go, 0)

    start_blk(0)

    def blk_loop(b, carry):
        @pl.when(b + 1 < NBLK)
        def _():
            start_blk(b + 1)

        def wait_b(j, c):
            chunk_copy(b * BLK + j, b & 1).wait()
            return c

        lax.fori_loop(0, BLK, wait_b, 0)
        per_quad(2 * b)
        per_quad(2 * b + 1)
        return carry

    lax.fori_loop(0, NBLK, blk_loop, 0)
    pltpu.sync_copy(o_v, out_hbm.at[pl.ds(wid * (BAGS_PER_W * NUM_Y),
                                          BAGS_PER_W * NUM_Y)])


def _sc_half_a(pA, idxA):
    mesh = plsc.VectorSubcoreMesh(core_axis_name="c", subcore_axis_name="s")
    kfn = functools.partial(
        pl.kernel,
        out_type=jax.ShapeDtypeStruct((BATCH * NUM_Y,), jnp.float32),
        mesh=mesh,
        scratch_types=[
            pltpu.VMEM((CHUNKS_PER_W, CHUNK), jnp.int32),
            pltpu.VMEM((IDX_PER_W,), jnp.int32),
            pltpu.VMEM((BAGS_PER_W * NUM_Y,), jnp.float32),
            pltpu.VMEM((16,), jnp.float32),
            pltpu.SemaphoreType.DMA((2,)),
        ],
        compiler_params=pltpu.CompilerParams(use_tc_tiling_on_sc=False,
                                             needs_layout_passes=False),
    )(functools.partial(_sc_body, False))
    return kfn(pA, idxA)


def _sc_half_b(pB, idxB, partA, bias16):
    mesh = plsc.VectorSubcoreMesh(core_axis_name="c", subcore_axis_name="s")
    kfn = functools.partial(
        pl.kernel,
        out_type=jax.ShapeDtypeStruct((BATCH * NUM_Y,), jnp.float32),
        mesh=mesh,
        scratch_types=[
            pltpu.VMEM((CHUNKS_PER_W, CHUNK), jnp.int32),
            pltpu.VMEM((IDX_PER_W,), jnp.int32),
            pltpu.VMEM((BAGS_PER_W * NUM_Y,), jnp.float32),
            pltpu.VMEM((BAGS_PER_W * NUM_Y,), jnp.float32),
            pltpu.VMEM((16,), jnp.float32),
            pltpu.SemaphoreType.DMA((2,)),
        ],
        compiler_params=pltpu.CompilerParams(use_tc_tiling_on_sc=False,
                                             needs_layout_passes=False),
    )(functools.partial(_sc_body, True))
    return kfn(pB, idxB, partA, bias16)


def kernel(input, emb_weight, W, b):
    idx = input.astype(jnp.int32).reshape(TOTAL_IDX // CHUNK, CHUNK)
    idxA = jnp.where(idx < HALF, idx, HALF)
    idxB = jnp.where(idx >= HALF, idx - HALF, DUMMY_B)
    embT = emb_weight.T
    pA = _proj_a(embT, W)
    pB = _proj_b(embT, W)
    bias16 = jnp.tile(b.astype(jnp.float32), 16 // NUM_Y)
    partA = _sc_half_a(pA, idxA)
    out = _sc_half_b(pB, idxB, partA, bias16)
    return out.reshape(BATCH, NUM_Y)


# speedup vs baseline: 27.3283x; 27.3283x over previous
import functools

import jax
import jax.numpy as jnp
from jax import lax
from jax.experimental import pallas as pl
from jax.experimental.pallas import tpu as pltpu
from jax.experimental.pallas import tpu_sc as plsc

VOCAB = 1000000
EMB_DIM = 64
NUM_Y = 2
BATCH = 4096
HIST = 200

NC = 2
NS = 16
NW = NC * NS

TOTAL_IDX = BATCH * HIST
IDX_PER_W = TOTAL_IDX // NW
CHUNK = 128
CHUNKS_PER_W = IDX_PER_W // CHUNK
BAGS_PER_W = BATCH // NW
GROUPS_PER_W = BAGS_PER_W // 2

PROJ_BLK = 65536
PROJ_GRID = -(-VOCAB // PROJ_BLK)


def _proj_body(embt_ref, w_ref, out_hbm, pk_v, sems):
    i = pl.program_id(0)
    grid = PROJ_GRID
    slot = lax.rem(i, 2)
    s = lax.dot_general(
        w_ref[...], embt_ref[...],
        (((1,), (0,)), ((), ())),
        preferred_element_type=jnp.float32,
    )
    s0 = s[0:1, :].astype(jnp.bfloat16)
    s1 = s[1:2, :].astype(jnp.bfloat16)
    u0 = lax.convert_element_type(
        lax.bitcast_convert_type(s0, jnp.uint16), jnp.uint32)
    u1 = lax.convert_element_type(
        lax.bitcast_convert_type(s1, jnp.uint16), jnp.uint32)
    packed = u0 | (u1 << 16)
    pk_v[pl.ds(slot, 1), :] = lax.bitcast_convert_type(packed, jnp.int32)

    def cp(j, sl):
        return pltpu.make_async_copy(
            pk_v.at[sl], out_hbm.at[pl.ds(j * PROJ_BLK, PROJ_BLK)],
            sems.at[sl])

    cp(i, slot).start()

    @pl.when(i > 0)
    def _():
        cp(i - 1, 1 - slot).wait()

    @pl.when(i == grid - 1)
    def _():
        cp(i, slot).wait()


def _project_pack(embT, W):
    grid = PROJ_GRID
    return pl.pallas_call(
        _proj_body,
        grid=(grid,),
        in_specs=[
            pl.BlockSpec((EMB_DIM, PROJ_BLK), lambda i: (0, i)),
            pl.BlockSpec((NUM_Y, EMB_DIM), lambda i: (0, 0)),
        ],
        out_specs=pl.BlockSpec(memory_space=pl.ANY),
        out_shape=jax.ShapeDtypeStruct((PROJ_GRID * PROJ_BLK,), jnp.int32),
        scratch_shapes=[pltpu.VMEM((2, PROJ_BLK), jnp.int32),
                        pltpu.SemaphoreType.DMA((2,))],
    )(embT, W)


def _sc_body(p_hbm, idx_hbm, bias_hbm, out_hbm, idx_v, g_v, o_v, b_v, sems):
    wid = lax.axis_index("s") * NC + lax.axis_index("c")
    pltpu.sync_copy(idx_hbm.at[pl.ds(wid * CHUNKS_PER_W, CHUNKS_PER_W)], idx_v)
    pltpu.sync_copy(bias_hbm, b_v)

    def chunk_copy(row, sl):
        return pltpu.make_async_copy(
            p_hbm.at[idx_v.at[row]],
            g_v.at[pl.ds(row * CHUNK, CHUNK)],
            sems.at[sl],
        )

    lanes = lax.iota(jnp.int32, 16)
    bias = b_v[...]
    zero = jnp.zeros((16,), jnp.float32)
    himask = jnp.int32(-65536)
    dnums = lax.GatherDimensionNumbers(
        offset_dims=(), collapsed_slice_dims=(0,), start_index_map=(0,))
    perms = [(lanes + step) % 16 for step in (8, 4, 2, 1)]

    def lane_perm(x, perm):
        return lax.gather(x, perm[:, None], dnums, (1,),
                          mode=lax.GatherScatterMode.PROMISE_IN_BOUNDS)

    def fold(x):
        for p in perms:
            x = x + lane_perm(x, p)
        return x

    def unpack(w):
        lo = plsc.bitcast(w << 16, jnp.float32)
        hi = plsc.bitcast(w & himask, jnp.float32)
        return lo, hi

    def per_quad(q, carry):
        out_acc = zero
        for t4 in range(4):
            t = q * 4 + t4
            base = t * (2 * HIST)

            def redA(k, ab):
                a0, a1 = ab
                lo, hi = unpack(g_v[pl.ds(base + k * 16, 16)])
                return a0 + lo, a1 + hi

            accA0, accA1 = lax.fori_loop(0, 12, redA, (zero, zero))
            lo, hi = unpack(g_v[pl.ds(base + 192, 16)])
            mA = lanes < 8
            accA0 = accA0 + jnp.where(mA, lo, zero)
            accA1 = accA1 + jnp.where(mA, hi, zero)
            accB0 = jnp.where(mA, zero, lo)
            accB1 = jnp.where(mA, zero, hi)

            def redB(k, ab):
                a0, a1 = ab
                lo, hi = unpack(g_v[pl.ds(base + k * 16, 16)])
                return a0 + lo, a1 + hi

            accB0, accB1 = lax.fori_loop(13, 25, redB, (accB0, accB1))

            sA0, sA1 = fold(accA0), fold(accA1)
            sB0, sB1 = fold(accB0), fold(accB1)
            qA = jnp.where((lanes & 1) == 0, sA0, sA1)
            qB = jnp.where((lanes & 1) == 0, sB0, sB1)
            quadv = jnp.where((lanes & 2) == 0, qA, qB)
            out_acc = jnp.where((lanes >> 2) == t4, quadv, out_acc)
        z = out_acc * (1.0 / HIST) + bias
        o_v[pl.ds(q * 16, 16)] = 1.0 / (1.0 + jnp.exp(-z))
        return carry

    BLK = 25
    NBLK = CHUNKS_PER_W // BLK

    def start_blk(b):
        def go(j, c):
            chunk_copy(b * BLK + j, b & 1).start()
            return c
        lax.fori_loop(0, BLK, go, 0)

    start_blk(0)

    def blk_loop(b, carry):
        @pl.when(b + 1 < NBLK)
        def _():
            start_blk(b + 1)

        def wait_b(j, c):
            chunk_copy(b * BLK + j, b & 1).wait()
            return c

        lax.fori_loop(0, BLK, wait_b, 0)
        per_quad(2 * b, 0)
        per_quad(2 * b + 1, 0)
        return carry

    lax.fori_loop(0, NBLK, blk_loop, 0)
    pltpu.sync_copy(o_v, out_hbm.at[pl.ds(wid * (BAGS_PER_W * NUM_Y),
                                          BAGS_PER_W * NUM_Y)])


def _sc_gather_pool(p_flat, idx2, bias16):
    mesh = plsc.VectorSubcoreMesh(core_axis_name="c", subcore_axis_name="s")
    kfn = functools.partial(
        pl.kernel,
        out_type=jax.ShapeDtypeStruct((BATCH * NUM_Y,), jnp.float32),
        mesh=mesh,
        scratch_types=[
            pltpu.VMEM((CHUNKS_PER_W, CHUNK), jnp.int32),
            pltpu.VMEM((IDX_PER_W,), jnp.int32),
            pltpu.VMEM((BAGS_PER_W * NUM_Y,), jnp.float32),
            pltpu.VMEM((16,), jnp.float32),
            pltpu.SemaphoreType.DMA((2,)),
        ],
        compiler_params=pltpu.CompilerParams(use_tc_tiling_on_sc=False,
                                             needs_layout_passes=False),
    )(_sc_body)
    return kfn(p_flat, idx2, bias16)


def kernel(input, emb_weight, W, b):
    idx2 = input.astype(jnp.int32).reshape(TOTAL_IDX // CHUNK, CHUNK)
    Pp = _project_pack(emb_weight.T, W)
    bias16 = jnp.tile(b.astype(jnp.float32), 16 // NUM_Y)
    out = _sc_gather_pool(Pp, idx2, bias16)
    return out.reshape(BATCH, NUM_Y)


# speedup vs baseline: 27.5625x; 1.0086x over previous
import functools

import jax
import jax.numpy as jnp
from jax import lax
from jax.experimental import pallas as pl
from jax.experimental.pallas import tpu as pltpu
from jax.experimental.pallas import tpu_sc as plsc

VOCAB = 1000000
EMB_DIM = 64
NUM_Y = 2
BATCH = 4096
HIST = 200

NC = 2
NS = 16
NW = NC * NS

TOTAL_IDX = BATCH * HIST
IDX_PER_W = TOTAL_IDX // NW
CHUNK = 128
CHUNKS_PER_W = IDX_PER_W // CHUNK
BAGS_PER_W = BATCH // NW
GROUPS_PER_W = BAGS_PER_W // 2

PROJ_BLK = 32768
PROJ_GRID = -(-VOCAB // PROJ_BLK)


def _proj_body(embt_ref, w_ref, out_hbm, pk_v, sems):
    i = pl.program_id(0)
    grid = PROJ_GRID
    slot = lax.rem(i, 2)
    s = lax.dot_general(
        w_ref[...], embt_ref[...],
        (((1,), (0,)), ((), ())),
        preferred_element_type=jnp.float32,
    )
    s0 = s[0:1, :].astype(jnp.bfloat16)
    s1 = s[1:2, :].astype(jnp.bfloat16)
    u0 = lax.convert_element_type(
        lax.bitcast_convert_type(s0, jnp.uint16), jnp.uint32)
    u1 = lax.convert_element_type(
        lax.bitcast_convert_type(s1, jnp.uint16), jnp.uint32)
    packed = u0 | (u1 << 16)
    pk_v[pl.ds(slot, 1), :] = lax.bitcast_convert_type(packed, jnp.int32)

    def cp(j, sl):
        return pltpu.make_async_copy(
            pk_v.at[sl], out_hbm.at[pl.ds(j * PROJ_BLK, PROJ_BLK)],
            sems.at[sl])

    cp(i, slot).start()

    @pl.when(i > 0)
    def _():
        cp(i - 1, 1 - slot).wait()

    @pl.when(i == grid - 1)
    def _():
        cp(i, slot).wait()


def _project_pack(embT, W):
    grid = PROJ_GRID
    return pl.pallas_call(
        _proj_body,
        grid=(grid,),
        in_specs=[
            pl.BlockSpec((EMB_DIM, PROJ_BLK), lambda i: (0, i)),
            pl.BlockSpec((NUM_Y, EMB_DIM), lambda i: (0, 0)),
        ],
        out_specs=pl.BlockSpec(memory_space=pl.ANY),
        out_shape=jax.ShapeDtypeStruct((PROJ_GRID * PROJ_BLK,), jnp.int32),
        scratch_shapes=[pltpu.VMEM((2, PROJ_BLK), jnp.int32),
                        pltpu.SemaphoreType.DMA((2,))],
    )(embT, W)


def _sc_body(p_hbm, idx_hbm, bias_hbm, out_hbm, idx_v, g_v, o_v, b_v, sems):
    wid = lax.axis_index("s") * NC + lax.axis_index("c")
    pltpu.sync_copy(idx_hbm.at[pl.ds(wid * CHUNKS_PER_W, CHUNKS_PER_W)], idx_v)
    pltpu.sync_copy(bias_hbm, b_v)

    def chunk_copy(row, sl):
        return pltpu.make_async_copy(
            p_hbm.at[idx_v.at[row]],
            g_v.at[pl.ds(row * CHUNK, CHUNK)],
            sems.at[sl],
        )

    lanes = lax.iota(jnp.int32, 16)
    bias = b_v[...]
    zero = jnp.zeros((16,), jnp.float32)
    himask = jnp.int32(-65536)
    dnums = lax.GatherDimensionNumbers(
        offset_dims=(), collapsed_slice_dims=(0,), start_index_map=(0,))
    perms = [(lanes + step) % 16 for step in (8, 4, 2, 1)]

    def lane_perm(x, perm):
        return lax.gather(x, perm[:, None], dnums, (1,),
                          mode=lax.GatherScatterMode.PROMISE_IN_BOUNDS)

    def fold(x):
        for p in perms:
            x = x + lane_perm(x, p)
        return x

    def unpack(w):
        lo = plsc.bitcast(w << 16, jnp.float32)
        hi = plsc.bitcast(w & himask, jnp.float32)
        return lo, hi

    def per_quad(q, carry):
        out_acc = zero
        for t4 in range(4):
            t = q * 4 + t4
            base = t * (2 * HIST)

            def redA(k, ab):
                a0, a1 = ab
                lo, hi = unpack(g_v[pl.ds(base + k * 16, 16)])
                return a0 + lo, a1 + hi

            accA0, accA1 = lax.fori_loop(0, 12, redA, (zero, zero))
            lo, hi = unpack(g_v[pl.ds(base + 192, 16)])
            mA = lanes < 8
            accA0 = accA0 + jnp.where(mA, lo, zero)
            accA1 = accA1 + jnp.where(mA, hi, zero)
            accB0 = jnp.where(mA, zero, lo)
            accB1 = jnp.where(mA, zero, hi)

            def redB(k, ab):
                a0, a1 = ab
                lo, hi = unpack(g_v[pl.ds(base + k * 16, 16)])
                return a0 + lo, a1 + hi

            accB0, accB1 = lax.fori_loop(13, 25, redB, (accB0, accB1))

            sA0, sA1 = fold(accA0), fold(accA1)
            sB0, sB1 = fold(accB0), fold(accB1)
            qA = jnp.where((lanes & 1) == 0, sA0, sA1)
            qB = jnp.where((lanes & 1) == 0, sB0, sB1)
            quadv = jnp.where((lanes & 2) == 0, qA, qB)
            out_acc = jnp.where((lanes >> 2) == t4, quadv, out_acc)
        z = out_acc * (1.0 / HIST) + bias
        o_v[pl.ds(q * 16, 16)] = 1.0 / (1.0 + jnp.exp(-z))
        return carry

    BLK = 25
    NBLK = CHUNKS_PER_W // BLK

    def start_blk(b):
        def go(j, c):
            chunk_copy(b * BLK + j, b & 1).start()
            return c
        lax.fori_loop(0, BLK, go, 0)

    start_blk(0)

    def blk_loop(b, carry):
        @pl.when(b + 1 < NBLK)
        def _():
            start_blk(b + 1)

        def wait_b(j, c):
            chunk_copy(b * BLK + j, b & 1).wait()
            return c

        lax.fori_loop(0, BLK, wait_b, 0)
        per_quad(2 * b, 0)
        per_quad(2 * b + 1, 0)
        return carry

    lax.fori_loop(0, NBLK, blk_loop, 0)
    pltpu.sync_copy(o_v, out_hbm.at[pl.ds(wid * (BAGS_PER_W * NUM_Y),
                                          BAGS_PER_W * NUM_Y)])


def _sc_gather_pool(p_flat, idx2, bias16):
    mesh = plsc.VectorSubcoreMesh(core_axis_name="c", subcore_axis_name="s")
    kfn = functools.partial(
        pl.kernel,
        out_type=jax.ShapeDtypeStruct((BATCH * NUM_Y,), jnp.float32),
        mesh=mesh,
        scratch_types=[
            pltpu.VMEM((CHUNKS_PER_W, CHUNK), jnp.int32),
            pltpu.VMEM((IDX_PER_W,), jnp.int32),
            pltpu.VMEM((BAGS_PER_W * NUM_Y,), jnp.float32),
            pltpu.VMEM((16,), jnp.float32),
            pltpu.SemaphoreType.DMA((2,)),
        ],
        compiler_params=pltpu.CompilerParams(use_tc_tiling_on_sc=False,
                                             needs_layout_passes=False),
    )(_sc_body)
    return kfn(p_flat, idx2, bias16)


def kernel(input, emb_weight, W, b):
    idx2 = input.astype(jnp.int32).reshape(TOTAL_IDX // CHUNK, CHUNK)
    Pp = _project_pack(emb_weight.T, W)
    bias16 = jnp.tile(b.astype(jnp.float32), 16 // NUM_Y)
    out = _sc_gather_pool(Pp, idx2, bias16)
    return out.reshape(BATCH, NUM_Y)
